# baseline (speedup 1.0000x reference)
import functools

import jax
import jax.numpy as jnp
from jax import lax
from jax.experimental import pallas as pl
from jax.experimental.pallas import tpu as pltpu
from jax.experimental.pallas import tpu_sc as plsc

B, L, D, K = 4, 8192, 768, 1024
NW = 32
RPW = (B * K) // NW
LANES = 16


CHUNK = 64
NCHUNKS = 4 * RPW // CHUNK


def _body(ta, tb, st, ia, ib, ica, icb, out_s, out_t, idx_v,
          buf0, buf1, g0, g1, w0, w1):
    wid = lax.axis_index("s") * 2 + lax.axis_index("c")
    flat_base = pl.multiple_of(wid * RPW, RPW)
    b = flat_base // K
    boff = b * L
    k_base = flat_base - b * K
    out_bb = b * (2 * K)

    jobs = (
        (ta, ia, out_t, 0),
        (tb, ib, out_t, K),
        (st, ica, out_s, 0),
        (st, icb, out_s, K),
    )
    cpj = RPW // CHUNK
    for j in range(4):
        iref = jobs[j][1]
        for c in range(cpj):
            pltpu.sync_copy(
                iref.at[pl.ds(flat_base + c * CHUNK, CHUNK)],
                idx_v.at[j * cpj + c])
    for r in range(NCHUNKS):
        for i in range(CHUNK // LANES):
            sl = pl.ds(i * LANES, LANES)
            idx_v[r, sl] = idx_v[r, sl] + boff

    chunks = []
    for j, (tab, _, oref, joff) in enumerate(jobs):
        for c in range(cpj):
            out_base = pl.multiple_of(out_bb + joff + k_base + c * CHUNK,
                                      CHUNK)
            chunks.append((tab, j * cpj + c, oref, out_base))

    bufs = (buf0, buf1)
    gsems = (g0, g1)
    wsems = (w0, w1)
    gathers = [None, None]
    writes = [None, None]
    for c, (tab, ioff, oref, out_base) in enumerate(chunks):
        s = c % 2
        if writes[s] is not None:
            writes[s].wait()
        gathers[s] = pltpu.async_copy(
            tab.at[idx_v.at[ioff]], bufs[s], gsems[s])
        if c == 0:
            continue
        sp = (c - 1) % 2
        gathers[sp].wait()
        writes[sp] = pltpu.async_copy(
            bufs[sp], chunks[c - 1][2].at[pl.ds(chunks[c - 1][3], CHUNK)],
            wsems[sp])
    sl = (NCHUNKS - 1) % 2
    gathers[sl].wait()
    writes[sl] = pltpu.async_copy(
        bufs[sl], chunks[-1][2].at[pl.ds(chunks[-1][3], CHUNK)], wsems[sl])
    writes[(NCHUNKS - 2) % 2].wait()
    writes[sl].wait()


@functools.partial(
    pl.kernel,
    mesh=plsc.VectorSubcoreMesh(core_axis_name="c", subcore_axis_name="s"),
    out_type=[
        jax.ShapeDtypeStruct((B * 2 * K, D), jnp.float32),
        jax.ShapeDtypeStruct((B * 2 * K, D), jnp.float32),
    ],
    scratch_types=[
        pltpu.VMEM((NCHUNKS, CHUNK), jnp.int32),
        pltpu.VMEM((CHUNK, D), jnp.float32),
        pltpu.VMEM((CHUNK, D), jnp.float32),
        pltpu.SemaphoreType.DMA,
        pltpu.SemaphoreType.DMA,
        pltpu.SemaphoreType.DMA,
        pltpu.SemaphoreType.DMA,
    ],
)
def _gather(ta, tb, st, ia, ib, ica, icb, out_s, out_t, idx_v,
            buf0, buf1, g0, g1, w0, w1):
    _body(ta, tb, st, ia, ib, ica, icb, out_s, out_t, idx_v,
          buf0, buf1, g0, g1, w0, w1)


def kernel(teacher_logits_a, teacher_logits_b, student_results,
           span_a_selected_index, span_b_selected_index,
           span_c_a_selected_index, span_c_b_selected_index):
    ta = teacher_logits_a.reshape(B * L, D)
    tb = teacher_logits_b.reshape(B * L, D)
    st = student_results.reshape(B * L, D)
    ia = span_a_selected_index.reshape(B * K).astype(jnp.int32)
    ib = span_b_selected_index.reshape(B * K).astype(jnp.int32)
    ica = span_c_a_selected_index.reshape(B * K).astype(jnp.int32)
    icb = span_c_b_selected_index.reshape(B * K).astype(jnp.int32)
    out_s, out_t = _gather(ta, tb, st, ia, ib, ica, icb)
    return (out_s.reshape(B, 2 * K, D), out_t.reshape(B, 2 * K, D))


# speedup vs baseline: 1.0335x; 1.0335x over previous
import functools

import jax
import jax.numpy as jnp
from jax import lax
from jax.experimental import pallas as pl
from jax.experimental.pallas import tpu as pltpu
from jax.experimental.pallas import tpu_sc as plsc

B, L, D, K = 4, 8192, 768, 1024
NW = 32
RPW = (B * K) // NW


def _body(ta, tb, st, ia, ib, ica, icb, out_s, out_t,
          i0, i1, i2, i3, rows_v, isem, gsem, wsem):
    wid = lax.axis_index("s") * 2 + lax.axis_index("c")
    flat_base = pl.multiple_of(wid * RPW, RPW)
    b = flat_base // K
    k_base = flat_base - b * K
    out_bb = b * (2 * K)

    jobs = (
        (ta, ia, out_t, 0, i0),
        (tb, ib, out_t, K, i1),
        (st, ica, out_s, 0, i2),
        (st, icb, out_s, K, i3),
    )
    icopies = [
        pltpu.async_copy(iref.at[b, pl.ds(k_base, RPW)], iv, isem)
        for (_, iref, _, _, iv) in jobs
    ]
    for c in icopies:
        c.wait()

    write = None
    for tab, _, oref, joff, iv in jobs:
        g = pltpu.async_copy(tab.at[b].at[iv], rows_v, gsem)
        if write is not None:
            write.wait()
        g.wait()
        out_base = pl.multiple_of(out_bb + joff + k_base, RPW)
        write = pltpu.async_copy(rows_v, oref.at[pl.ds(out_base, RPW)], wsem)
    write.wait()


@functools.partial(
    pl.kernel,
    mesh=plsc.VectorSubcoreMesh(core_axis_name="c", subcore_axis_name="s"),
    out_type=[
        jax.ShapeDtypeStruct((B * 2 * K, D), jnp.float32),
        jax.ShapeDtypeStruct((B * 2 * K, D), jnp.float32),
    ],
    scratch_types=[
        pltpu.VMEM((RPW,), jnp.int32),
        pltpu.VMEM((RPW,), jnp.int32),
        pltpu.VMEM((RPW,), jnp.int32),
        pltpu.VMEM((RPW,), jnp.int32),
        pltpu.VMEM((RPW, D), jnp.float32),
        pltpu.SemaphoreType.DMA,
        pltpu.SemaphoreType.DMA,
        pltpu.SemaphoreType.DMA,
    ],
)
def _gather(ta, tb, st, ia, ib, ica, icb, out_s, out_t,
            i0, i1, i2, i3, rows_v, isem, gsem, wsem):
    _body(ta, tb, st, ia, ib, ica, icb, out_s, out_t,
          i0, i1, i2, i3, rows_v, isem, gsem, wsem)


def kernel(teacher_logits_a, teacher_logits_b, student_results,
           span_a_selected_index, span_b_selected_index,
           span_c_a_selected_index, span_c_b_selected_index):
    out_s, out_t = _gather(
        teacher_logits_a, teacher_logits_b, student_results,
        span_a_selected_index.astype(jnp.int32),
        span_b_selected_index.astype(jnp.int32),
        span_c_a_selected_index.astype(jnp.int32),
        span_c_b_selected_index.astype(jnp.int32))
    return (out_s.reshape(B, 2 * K, D), out_t.reshape(B, 2 * K, D))


# speedup vs baseline: 1.0448x; 1.0108x over previous
import functools

import jax
import jax.numpy as jnp
from jax import lax
from jax.experimental import pallas as pl
from jax.experimental.pallas import tpu as pltpu
from jax.experimental.pallas import tpu_sc as plsc

B, L, D, K = 4, 8192, 768, 1024
NW = 32
RPW = (B * K) // NW


def _body(ta, tb, st, ia, ib, ica, icb, out_s, out_t,
          i0, i1, i2, i3, rows_v, isem, gsem, wsem):
    wid = lax.axis_index("s") * 2 + lax.axis_index("c")
    flat_base = pl.multiple_of(wid * RPW, RPW)
    b = flat_base // K
    k_base = flat_base - b * K
    out_bb = b * (2 * K)

    jobs = (
        (ta, ia, out_t, 0, i0),
        (tb, ib, out_t, K, i1),
        (st, ica, out_s, 0, i2),
        (st, icb, out_s, K, i3),
    )
    icopies = [
        pltpu.async_copy(iref.at[b, pl.ds(k_base, RPW)], iv, isem)
        for (_, iref, _, _, iv) in jobs
    ]
    for c in icopies:
        c.wait()

    write = None
    for tab, _, oref, joff, iv in jobs:
        if write is not None:
            write.wait()
        g = pltpu.async_copy(tab.at[b].at[iv], rows_v, gsem)
        g.wait()
        out_base = pl.multiple_of(out_bb + joff + k_base, RPW)
        write = pltpu.async_copy(rows_v, oref.at[pl.ds(out_base, RPW)], wsem)
    write.wait()


@functools.partial(
    pl.kernel,
    mesh=plsc.VectorSubcoreMesh(core_axis_name="c", subcore_axis_name="s"),
    out_type=[
        jax.ShapeDtypeStruct((B * 2 * K, D), jnp.float32),
        jax.ShapeDtypeStruct((B * 2 * K, D), jnp.float32),
    ],
    scratch_types=[
        pltpu.VMEM((RPW,), jnp.int32),
        pltpu.VMEM((RPW,), jnp.int32),
        pltpu.VMEM((RPW,), jnp.int32),
        pltpu.VMEM((RPW,), jnp.int32),
        pltpu.VMEM((RPW, D), jnp.float32),
        pltpu.SemaphoreType.DMA,
        pltpu.SemaphoreType.DMA,
        pltpu.SemaphoreType.DMA,
    ],
)
def _gather(ta, tb, st, ia, ib, ica, icb, out_s, out_t,
            i0, i1, i2, i3, rows_v, isem, gsem, wsem):
    _body(ta, tb, st, ia, ib, ica, icb, out_s, out_t,
          i0, i1, i2, i3, rows_v, isem, gsem, wsem)


def kernel(teacher_logits_a, teacher_logits_b, student_results,
           span_a_selected_index, span_b_selected_index,
           span_c_a_selected_index, span_c_b_selected_index):
    out_s, out_t = _gather(
        teacher_logits_a, teacher_logits_b, student_results,
        span_a_selected_index.astype(jnp.int32),
        span_b_selected_index.astype(jnp.int32),
        span_c_a_selected_index.astype(jnp.int32),
        span_c_b_selected_index.astype(jnp.int32))
    return (out_s.reshape(B, 2 * K, D), out_t.reshape(B, 2 * K, D))
